# baseline (speedup 1.0000x reference)
import functools

import jax
import jax.numpy as jnp
from jax import lax
from jax.experimental import pallas as pl
from jax.experimental.pallas import tpu as pltpu
from jax.experimental.pallas import tpu_sc as plsc

B = 16384
D = 64
NC = 2
NS = 16
NW = NC * NS
BPW = B // NW
CHUNK = 128
NCHUNK = BPW // CHUNK

_GATHER_DNUMS = lax.GatherDimensionNumbers(
    offset_dims=(), collapsed_slice_dims=(0,), start_index_map=(0,))


def _shuffle(x, idx):
    return lax.gather(x, idx[:, None], _GATHER_DNUMS, slice_sizes=(1,),
                      mode=lax.GatherScatterMode.PROMISE_IN_BOUNDS)


def _sc_body(uidx_hbm, pidx_hbm, uw_hbm, pw_hbm, out_hbm,
             uidx_v, pidx_v, urows_v, prows_v, out_v, sem):
    wid = lax.axis_index("s") * NC + lax.axis_index("c")
    base = wid * BPW

    pltpu.sync_copy(uidx_hbm.at[pl.ds(wid * NCHUNK, NCHUNK)], uidx_v)
    pltpu.sync_copy(pidx_hbm.at[pl.ds(wid * NCHUNK, NCHUNK)], pidx_v)

    copies = []
    for j in range(NCHUNK):
        copies.append(pltpu.async_copy(
            uw_hbm.at[uidx_v.at[j]], urows_v.at[pl.ds(j * CHUNK, CHUNK)], sem))
        copies.append(pltpu.async_copy(
            pw_hbm.at[pidx_v.at[j]], prows_v.at[pl.ds(j * CHUNK, CHUNK)], sem))
    for c in copies:
        c.wait()

    lane = lax.iota(jnp.int32, 16)

    def g_body(g, carry):
        out_vec = jnp.zeros((16,), jnp.float32)
        for r in range(16):
            row = g * 16 + r
            acc = urows_v[row, pl.ds(0, 16)] * prows_v[row, pl.ds(0, 16)]
            for dd in range(1, D // 16):
                acc = acc + (urows_v[row, pl.ds(dd * 16, 16)]
                             * prows_v[row, pl.ds(dd * 16, 16)])
            for sh in (8, 4, 2, 1):
                acc = acc + _shuffle(acc, lane ^ sh)
            out_vec = jnp.where(lane == r, acc, out_vec)
        out_v[pl.ds(g * 16, 16)] = out_vec
        return carry

    lax.fori_loop(0, BPW // 16, g_body, 0)
    pltpu.sync_copy(out_v, out_hbm.at[pl.ds(base, BPW)])


@jax.jit
def _collab_dot(uidx, pidx, users_w, products_w):
    run = functools.partial(
        pl.kernel,
        mesh=plsc.VectorSubcoreMesh(core_axis_name="c", subcore_axis_name="s"),
        compiler_params=pltpu.CompilerParams(use_tc_tiling_on_sc=False),
        out_type=jax.ShapeDtypeStruct((B,), jnp.float32),
        scratch_types=[
            pltpu.VMEM((NCHUNK, CHUNK), jnp.int32),
            pltpu.VMEM((NCHUNK, CHUNK), jnp.int32),
            pltpu.VMEM((BPW, D), jnp.float32),
            pltpu.VMEM((BPW, D), jnp.float32),
            pltpu.VMEM((BPW,), jnp.float32),
            pltpu.SemaphoreType.DMA,
        ],
    )(_sc_body)
    return run(uidx, pidx, users_w, products_w)


def kernel(inputs, users_w, products_w):
    uidx = inputs[:, 0].reshape(B // CHUNK, CHUNK)
    pidx = inputs[:, 1].reshape(B // CHUNK, CHUNK)
    out = _collab_dot(uidx, pidx, users_w, products_w)
    return out[:, None]
